# baseline (speedup 1.0000x reference)
import functools

import jax
import jax.numpy as jnp
from jax import lax
from jax.experimental import pallas as pl
from jax.experimental.pallas import tpu as pltpu
from jax.experimental.pallas import tpu_sc as plsc

N_DIM = 4096
T_OUT = 1000000
NW = 32
B_PER_W = 31296
T_PAD = NW * B_PER_W
N_CHUNKS = 4
B_C = B_PER_W // N_CHUNKS
V_ITERS = B_C // 16


def _build_sc_kernel():
  mesh = plsc.VectorSubcoreMesh(core_axis_name="c", subcore_axis_name="s")

  @functools.partial(
      pl.kernel,
      mesh=mesh,
      out_type=jax.ShapeDtypeStruct((T_PAD,), jnp.float32),
      scratch_types=[
          pltpu.VMEM((B_C,), jnp.int32),
          pltpu.VMEM((B_C,), jnp.int32),
          pltpu.VMEM((B_C,), jnp.int32),
          pltpu.VMEM((B_C,), jnp.int32),
          pltpu.VMEM((B_C,), jnp.int32),
          pltpu.VMEM((B_C,), jnp.int32),
          pltpu.VMEM((B_C,), jnp.float32),
          pltpu.VMEM((B_C,), jnp.float32),
          pltpu.VMEM((B_C,), jnp.float32),
          pltpu.VMEM((B_C,), jnp.float32),
          pltpu.SemaphoreType.DMA,
      ],
  )
  def tri_gather(a_hbm, i_hbm, j_hbm, k_hbm, out_hbm,
                 iv, jv, kv, ij, ik, jk, vij, vik, vjk, ov, sem):
    wid = lax.axis_index("s") * 2 + lax.axis_index("c")
    wbase = wid * B_PER_W

    def chunk_body(c, carry):
      base = wbase + c * B_C
      pltpu.sync_copy(i_hbm.at[pl.ds(base, B_C)], iv)
      pltpu.sync_copy(j_hbm.at[pl.ds(base, B_C)], jv)
      pltpu.sync_copy(k_hbm.at[pl.ds(base, B_C)], kv)

      def idx_body(t, carry2):
        s = pl.ds(t * 16, 16)
        a = iv[s]
        b = jv[s]
        cc = kv[s]
        ij[s] = a * N_DIM + b
        ik[s] = a * N_DIM + cc
        jk[s] = b * N_DIM + cc
        return carry2

      lax.fori_loop(0, V_ITERS, idx_body, 0)

      cp1 = pltpu.async_copy(a_hbm.at[ij], vij, sem)
      cp2 = pltpu.async_copy(a_hbm.at[ik], vik, sem)
      cp3 = pltpu.async_copy(a_hbm.at[jk], vjk, sem)
      cp1.wait()
      cp2.wait()
      cp3.wait()

      def avg_body(t, carry2):
        s = pl.ds(t * 16, 16)
        ov[s] = (vij[s] + vik[s] + vjk[s]) * (1.0 / 3.0)
        return carry2

      lax.fori_loop(0, V_ITERS, avg_body, 0)
      pltpu.sync_copy(ov, out_hbm.at[pl.ds(base, B_C)])
      return carry

    lax.fori_loop(0, N_CHUNKS, chunk_body, 0)

  return tri_gather


_tri_gather = _build_sc_kernel()


@jax.jit
def kernel(A_s, triangles_indexes):
  a_flat = A_s.reshape(-1)
  pad = T_PAD - T_OUT
  i_idx = jnp.pad(triangles_indexes[:, 0], (0, pad))
  j_idx = jnp.pad(triangles_indexes[:, 1], (0, pad))
  k_idx = jnp.pad(triangles_indexes[:, 2], (0, pad))
  out = _tri_gather(a_flat, i_idx, j_idx, k_idx)
  return out[:T_OUT]


# speedup vs baseline: 1.0864x; 1.0864x over previous
import functools

import jax
import jax.numpy as jnp
from jax import lax
from jax.experimental import pallas as pl
from jax.experimental.pallas import tpu as pltpu
from jax.experimental.pallas import tpu_sc as plsc

N_DIM = 4096
T_OUT = 1000000
NW = 32
B_PER_W = 31296
T_PAD = NW * B_PER_W
N_CHUNKS = 6
B_C = B_PER_W // N_CHUNKS
UNROLL = 8


def _build_sc_kernel():
  mesh = plsc.VectorSubcoreMesh(core_axis_name="c", subcore_axis_name="s")

  idx_t = pltpu.VMEM((B_C,), jnp.int32)
  val_t = pltpu.VMEM((B_C,), jnp.float32)

  @functools.partial(
      pl.kernel,
      mesh=mesh,
      out_type=jax.ShapeDtypeStruct((T_PAD,), jnp.float32),
      scratch_types=[
          idx_t, idx_t, idx_t,
          [idx_t, idx_t, idx_t],
          [idx_t, idx_t, idx_t],
          [val_t, val_t, val_t],
          [val_t, val_t, val_t],
          val_t,
          pltpu.SemaphoreType.DMA,
          pltpu.SemaphoreType.DMA,
      ],
  )
  def tri_gather(a_hbm, i_hbm, j_hbm, k_hbm, out_hbm,
                 iv, jv, kv, idx0, idx1, val0, val1, ov, sem0, sem1):
    wid = lax.axis_index("s") * 2 + lax.axis_index("c")
    wbase = wid * B_PER_W
    idx_sets = (idx0, idx1)
    val_sets = (val0, val1)
    sems = (sem0, sem1)

    def copy_raw(c):
      base = wbase + c * B_C
      pltpu.sync_copy(i_hbm.at[pl.ds(base, B_C)], iv)
      pltpu.sync_copy(j_hbm.at[pl.ds(base, B_C)], jv)
      pltpu.sync_copy(k_hbm.at[pl.ds(base, B_C)], kv)

    def compute_idx(p):
      ij, ik, jk = idx_sets[p]

      @plsc.parallel_loop(0, B_C, step=16, unroll=UNROLL)
      def _(t):
        s = pl.ds(t, 16)
        a = iv[s]
        b = jv[s]
        cc = kv[s]
        ij[s] = a * N_DIM + b
        ik[s] = a * N_DIM + cc
        jk[s] = b * N_DIM + cc

    def fire(p):
      ij, ik, jk = idx_sets[p]
      vij, vik, vjk = val_sets[p]
      return (pltpu.async_copy(a_hbm.at[ij], vij, sems[p]),
              pltpu.async_copy(a_hbm.at[ik], vik, sems[p]),
              pltpu.async_copy(a_hbm.at[jk], vjk, sems[p]))

    def avg_out(c, p, cps):
      for cp in cps:
        cp.wait()
      vij, vik, vjk = val_sets[p]

      @plsc.parallel_loop(0, B_C, step=16, unroll=UNROLL)
      def _(t):
        s = pl.ds(t, 16)
        ov[s] = (vij[s] + vik[s] + vjk[s]) * (1.0 / 3.0)

      base = wbase + c * B_C
      pltpu.sync_copy(ov, out_hbm.at[pl.ds(base, B_C)])

    copy_raw(0)
    compute_idx(0)
    cps = {0: fire(0)}
    for c in range(1, N_CHUNKS):
      p = c % 2
      copy_raw(c)
      compute_idx(p)
      cps[p] = fire(p)
      avg_out(c - 1, 1 - p, cps[1 - p])
    last = N_CHUNKS - 1
    avg_out(last, last % 2, cps[last % 2])

  return tri_gather


_tri_gather = _build_sc_kernel()


@jax.jit
def kernel(A_s, triangles_indexes):
  a_flat = A_s.reshape(-1)
  pad = T_PAD - T_OUT
  i_idx = jnp.pad(triangles_indexes[:, 0], (0, pad))
  j_idx = jnp.pad(triangles_indexes[:, 1], (0, pad))
  k_idx = jnp.pad(triangles_indexes[:, 2], (0, pad))
  out = _tri_gather(a_flat, i_idx, j_idx, k_idx)
  return out[:T_OUT]


# speedup vs baseline: 1.0878x; 1.0013x over previous
import functools

import jax
import jax.numpy as jnp
from jax import lax
from jax.experimental import pallas as pl
from jax.experimental.pallas import tpu as pltpu
from jax.experimental.pallas import tpu_sc as plsc

N_DIM = 4096
T_OUT = 1000000
NW = 32
B_PER_W = 31296
T_PAD = NW * B_PER_W
N_CHUNKS = 6
B_C = B_PER_W // N_CHUNKS
UNROLL = 8


def _build_sc_kernel():
  mesh = plsc.VectorSubcoreMesh(core_axis_name="c", subcore_axis_name="s")

  idx_t = pltpu.VMEM((B_C,), jnp.int32)
  idx3_t = pltpu.VMEM((3 * B_C,), jnp.int32)
  val3_t = pltpu.VMEM((3 * B_C,), jnp.float32)
  val_t = pltpu.VMEM((B_C,), jnp.float32)

  @functools.partial(
      pl.kernel,
      mesh=mesh,
      out_type=jax.ShapeDtypeStruct((T_PAD,), jnp.float32),
      scratch_types=[
          idx_t, idx_t, idx_t,
          idx3_t, idx3_t,
          val3_t, val3_t,
          val_t,
          pltpu.SemaphoreType.DMA,
          pltpu.SemaphoreType.DMA,
      ],
  )
  def tri_gather(a_hbm, i_hbm, j_hbm, k_hbm, out_hbm,
                 iv, jv, kv, idx0, idx1, val0, val1, ov, sem0, sem1):
    wid = lax.axis_index("s") * 2 + lax.axis_index("c")
    wbase = wid * B_PER_W
    idx_sets = (idx0, idx1)
    val_sets = (val0, val1)
    sems = (sem0, sem1)

    def copy_raw(c):
      base = wbase + c * B_C
      pltpu.sync_copy(i_hbm.at[pl.ds(base, B_C)], iv)
      pltpu.sync_copy(j_hbm.at[pl.ds(base, B_C)], jv)
      pltpu.sync_copy(k_hbm.at[pl.ds(base, B_C)], kv)

    def compute_idx(p):
      idx = idx_sets[p]

      @plsc.parallel_loop(0, B_C, step=16, unroll=UNROLL)
      def _(t):
        s = pl.ds(t, 16)
        a = iv[s]
        b = jv[s]
        cc = kv[s]
        an = a * N_DIM
        idx[s] = an + b
        idx[pl.ds(t + B_C, 16)] = an + cc
        idx[pl.ds(t + 2 * B_C, 16)] = b * N_DIM + cc

    def fire(p):
      return pltpu.async_copy(a_hbm.at[idx_sets[p]], val_sets[p], sems[p])

    def avg_out(c, p, cp):
      cp.wait()
      val = val_sets[p]

      @plsc.parallel_loop(0, B_C, step=16, unroll=UNROLL)
      def _(t):
        ov[pl.ds(t, 16)] = (
            val[pl.ds(t, 16)]
            + val[pl.ds(t + B_C, 16)]
            + val[pl.ds(t + 2 * B_C, 16)]
        ) * (1.0 / 3.0)

      base = wbase + c * B_C
      pltpu.sync_copy(ov, out_hbm.at[pl.ds(base, B_C)])

    copy_raw(0)
    compute_idx(0)
    cps = {0: fire(0)}
    for c in range(1, N_CHUNKS):
      p = c % 2
      copy_raw(c)
      compute_idx(p)
      cps[p] = fire(p)
      avg_out(c - 1, 1 - p, cps[1 - p])
    last = N_CHUNKS - 1
    avg_out(last, last % 2, cps[last % 2])

  return tri_gather


_tri_gather = _build_sc_kernel()


@jax.jit
def kernel(A_s, triangles_indexes):
  a_flat = A_s.reshape(-1)
  pad = T_PAD - T_OUT
  i_idx = jnp.pad(triangles_indexes[:, 0], (0, pad))
  j_idx = jnp.pad(triangles_indexes[:, 1], (0, pad))
  k_idx = jnp.pad(triangles_indexes[:, 2], (0, pad))
  out = _tri_gather(a_flat, i_idx, j_idx, k_idx)
  return out[:T_OUT]
